# baseline (speedup 1.0000x reference)
import jax
import jax.numpy as jnp
from jax import lax
from jax.experimental import pallas as pl
from jax.experimental.pallas import tpu as pltpu
from jax.experimental.pallas import tpu_sc as plsc

EMBED = 64
NC, NS = 2, 16
NW = NC * NS
IDXW = 128
CHUNK = 1024
GROUPS = CHUNK // IDXW


def _sc_gather(table, idx2d):
    num_indices = idx2d.shape[0] * idx2d.shape[1]
    rows_per_w = num_indices // NW
    chunks_per_w = rows_per_w // CHUNK
    mesh = plsc.VectorSubcoreMesh(core_axis_name="c", subcore_axis_name="s")

    @pl.kernel(
        out_type=jax.ShapeDtypeStruct((num_indices, EMBED), table.dtype),
        mesh=mesh,
        scratch_types=[
            pltpu.VMEM((GROUPS, IDXW), jnp.int32),
            pltpu.VMEM((CHUNK, EMBED), jnp.float32),
            pltpu.SemaphoreType.DMA,
        ],
        compiler_params=pltpu.CompilerParams(use_tc_tiling_on_sc=False),
    )
    def emb_gather(table_hbm, idx_hbm, out_hbm, idx_v, rows_v, sem):
        wid = lax.axis_index("s") * NC + lax.axis_index("c")

        @pl.loop(0, chunks_per_w)
        def _(c):
            row0 = pl.multiple_of(wid * rows_per_w + c * CHUNK, CHUNK)
            grp0 = pl.multiple_of(row0 // IDXW, GROUPS)
            pltpu.sync_copy(idx_hbm.at[pl.ds(grp0, GROUPS)], idx_v)
            handles = [
                pltpu.async_copy(
                    table_hbm.at[idx_v.at[g]],
                    rows_v.at[pl.ds(g * IDXW, IDXW)],
                    sem,
                )
                for g in range(GROUPS)
            ]
            for h in handles:
                h.wait()
            pltpu.sync_copy(rows_v, out_hbm.at[pl.ds(row0, CHUNK)])

    return emb_gather(table, idx2d)


@jax.jit
def kernel(sequence, table):
    b, s = sequence.shape
    idx2d = sequence.reshape(b * s // IDXW, IDXW).astype(jnp.int32)
    out = _sc_gather(table, idx2d)
    return out.reshape(b, s, EMBED)


# speedup vs baseline: 1.0150x; 1.0150x over previous
import jax
import jax.numpy as jnp
from jax import lax
from jax.experimental import pallas as pl
from jax.experimental.pallas import tpu as pltpu
from jax.experimental.pallas import tpu_sc as plsc

EMBED = 64
NC, NS = 2, 16
NW = NC * NS
IDXW = 128
CHUNK = 640
GROUPS = CHUNK // IDXW


def _sc_gather(table, idx2d):
    num_indices = idx2d.shape[0] * idx2d.shape[1]
    rows_per_w = num_indices // NW
    grps_per_w = rows_per_w // IDXW
    nchunks = rows_per_w // CHUNK
    mesh = plsc.VectorSubcoreMesh(core_axis_name="c", subcore_axis_name="s")

    @pl.kernel(
        out_type=jax.ShapeDtypeStruct((num_indices, EMBED), table.dtype),
        mesh=mesh,
        scratch_types=[
            pltpu.VMEM((grps_per_w, IDXW), jnp.int32),
            pltpu.VMEM((CHUNK, EMBED), jnp.float32),
            pltpu.VMEM((CHUNK, EMBED), jnp.float32),
            pltpu.SemaphoreType.DMA,
            pltpu.SemaphoreType.DMA,
            pltpu.SemaphoreType.DMA,
            pltpu.SemaphoreType.DMA,
        ],
        compiler_params=pltpu.CompilerParams(use_tc_tiling_on_sc=False),
    )
    def emb_gather(table_hbm, idx_hbm, out_hbm, idx_v, rows_a, rows_b,
                   sem_ga, sem_gb, sem_wa, sem_wb):
        wid = lax.axis_index("s") * NC + lax.axis_index("c")
        grp0 = pl.multiple_of(wid * grps_per_w, 8)
        pltpu.sync_copy(idx_hbm.at[pl.ds(grp0, grps_per_w)], idx_v)

        rows = (rows_a, rows_b)
        sem_g = (sem_ga, sem_gb)
        sem_w = (sem_wa, sem_wb)

        def fire_gather(c, b):
            for g in range(GROUPS):
                pltpu.async_copy(
                    table_hbm.at[idx_v.at[c * GROUPS + g]],
                    rows[b].at[pl.ds(g * IDXW, IDXW)],
                    sem_g[b],
                )

        def fire_writeout(c, b):
            row0 = wid * rows_per_w + c * CHUNK
            pltpu.async_copy(rows[b], out_hbm.at[pl.ds(row0, CHUNK)],
                             sem_w[b])

        def drain_gather(b):
            pltpu.make_async_copy(table_hbm.at[pl.ds(0, CHUNK)], rows[b],
                                  sem_g[b]).wait()

        def drain_writeout(b):
            pltpu.make_async_copy(rows[b], out_hbm.at[pl.ds(0, CHUNK)],
                                  sem_w[b]).wait()

        fire_gather(0, 0)

        @pl.loop(0, nchunks - 2, step=2)
        def _(c):
            for b in range(2):
                fire_gather(c + 1 + b, 1 - b)
                drain_gather(b)
                fire_writeout(c + b, b)
                drain_writeout(b)

        fire_gather(nchunks - 1, 1)
        drain_gather(0)
        fire_writeout(nchunks - 2, 0)
        drain_gather(1)
        fire_writeout(nchunks - 1, 1)
        drain_writeout(0)
        drain_writeout(1)

    return emb_gather(table, idx2d)


@jax.jit
def kernel(sequence, table):
    b, s = sequence.shape
    idx2d = sequence.reshape(b * s // IDXW, IDXW).astype(jnp.int32)
    out = _sc_gather(table, idx2d)
    return out.reshape(b, s, EMBED)
